# baseline (speedup 1.0000x reference)
import jax
import jax.numpy as jnp
from jax import lax
from jax.experimental import pallas as pl
from jax.experimental.pallas import tpu as pltpu
from jax.experimental.pallas import tpu_sc as plsc

N_NODES = 10000
N_EDGES = 320000
CHANNELS = 128
EPSILON = 0.1
GAMMA = 0.1

NPAD = 10240
NC = 2
NS = 16
NW = NC * NS
CH = 80
EPW = N_EDGES // NW
M = EPW // CH
ROWS_PER_SUB = NPAD // NS


def _copy80(src_ref, dst_ref, base):
    for k in range(CH // 16):
        dst_ref[pl.ds(k * 16, 16)] = src_ref[pl.ds(base + k * 16, 16)]


def _degree_body(edge_hbm, z_hbm, deg_out, deg_sp, ones_v, didx1d,
                 dcur0, dcur1, sem0, sem1):
    c = lax.axis_index("c")
    s = lax.axis_index("s")
    wid = c * NS + s

    pltpu.sync_copy(z_hbm, deg_sp.at[pl.ds(s * ROWS_PER_SUB, ROWS_PER_SUB)])
    pltpu.sync_copy(edge_hbm.at[NW + wid], didx1d)

    ov = jnp.ones((16,), jnp.float32)

    def fill_ones(i, _):
        ones_v[i, :] = ov
        return 0

    lax.fori_loop(0, CH, fill_ones, 0)
    plsc.subcore_barrier()

    _copy80(didx1d, dcur0, 0)
    pltpu.async_copy(ones_v, deg_sp.at[dcur0], sem0, add=True)
    _copy80(didx1d, dcur1, CH)
    pltpu.async_copy(ones_v, deg_sp.at[dcur1], sem1, add=True)

    def body(jj, _):
        a = 2 * jj
        b = a + 1
        pltpu.make_async_copy(ones_v, deg_sp.at[dcur0], sem0).wait()
        _copy80(didx1d, dcur0, (a + 2) * CH)
        pltpu.async_copy(ones_v, deg_sp.at[dcur0], sem0, add=True)
        pltpu.make_async_copy(ones_v, deg_sp.at[dcur1], sem1).wait()

        @pl.when(b + 2 < M)
        def _():
            _copy80(didx1d, dcur1, (b + 2) * CH)
            pltpu.async_copy(ones_v, deg_sp.at[dcur1], sem1, add=True)

        return 0

    lax.fori_loop(0, (M - 1) // 2, body, 0)
    pltpu.make_async_copy(ones_v, deg_sp.at[dcur0], sem0).wait()
    plsc.subcore_barrier()

    sl = pl.ds(s * ROWS_PER_SUB, ROWS_PER_SUB)
    pltpu.sync_copy(deg_sp.at[sl], deg_out.at[c, sl])


NSLOT = 4


def _aggregate_body(edge_hbm, y_hbm, z_hbm, props_out,
                    acc_sp, sidx1d, didx1d, dcurs, rows, sems):
    c = lax.axis_index("c")
    s = lax.axis_index("s")
    wid = c * NS + s

    pltpu.sync_copy(z_hbm, acc_sp.at[pl.ds(s * ROWS_PER_SUB, ROWS_PER_SUB)])
    pltpu.sync_copy(edge_hbm.at[wid], sidx1d)
    pltpu.sync_copy(edge_hbm.at[NW + wid], didx1d)
    plsc.subcore_barrier()

    def g_idx(j):
        return sidx1d.at[pl.ds(j * CH, CH)]

    for k in range(NSLOT):
        pltpu.async_copy(y_hbm.at[g_idx(k)], rows[k], sems[k])

    def body(jj, _):
        base4 = NSLOT * jj
        for k in range(NSLOT):
            ch = base4 + k
            _copy80(didx1d, dcurs[k], ch * CH)
            pltpu.make_async_copy(y_hbm.at[g_idx(ch)], rows[k], sems[k]).wait()
            pltpu.sync_copy(rows[k], acc_sp.at[dcurs[k]], add=True)

            @pl.when(ch + NSLOT < M)
            def _():
                pltpu.async_copy(y_hbm.at[g_idx(ch + NSLOT)], rows[k], sems[k])

        return 0

    lax.fori_loop(0, (M - 1) // NSLOT, body, 0)
    _copy80(didx1d, dcurs[0], (M - 1) * CH)
    pltpu.make_async_copy(y_hbm.at[g_idx(M - 1)], rows[0], sems[0]).wait()
    pltpu.sync_copy(rows[0], acc_sp.at[dcurs[0]], add=True)
    plsc.subcore_barrier()

    sl = pl.ds(s * ROWS_PER_SUB, ROWS_PER_SUB)
    pltpu.sync_copy(acc_sp.at[sl], props_out.at[c * NS + s])


def _prepass_body(x_ref, deg_ref, y_ref):
    deg = (deg_ref[0] + deg_ref[1])[:, 0:1]
    dinv = jnp.where(deg > 0.5, lax.rsqrt(jnp.maximum(deg, 1e-12)), 0.0)
    y_ref[...] = (x_ref[...] * dinv).astype(jnp.bfloat16)


def _final_body(x_ref, w_ref, b_ref, deg_ref, p0_ref, p1_ref, out_ref):
    deg = (deg_ref[0] + deg_ref[1])[:, 0:1]
    dinv = jnp.where(deg > 0.5, lax.rsqrt(jnp.maximum(deg, 1e-12)), 0.0)
    blk = out_ref.shape[0]
    prop = (p0_ref[...].astype(jnp.float32).reshape(blk, CHANNELS)
            + p1_ref[...].astype(jnp.float32).reshape(blk, CHANNELS)) * dinv
    w = w_ref[...]
    ii = lax.broadcasted_iota(jnp.int32, (CHANNELS, CHANNELS), 0)
    jj = lax.broadcasted_iota(jnp.int32, (CHANNELS, CHANNELS), 1)
    eye = jnp.where(ii == jj, jnp.float32(GAMMA), jnp.float32(0.0))
    op = w - w.T - eye
    xb = x_ref[...]
    pre = jnp.dot(xb, op, preferred_element_type=jnp.float32) + b_ref[...]
    out_ref[...] = xb + EPSILON * jnp.tanh(pre + prop)


def kernel(x, edge_index, weight, bias):
    edge2 = edge_index.astype(jnp.int32).reshape(2 * NW, EPW)
    bias2d = bias.reshape(1, CHANNELS)
    z16 = jnp.zeros((ROWS_PER_SUB, 16), jnp.float32)
    z128 = jnp.zeros((ROWS_PER_SUB, CHANNELS), jnp.bfloat16)

    mesh = plsc.VectorSubcoreMesh(core_axis_name="c", subcore_axis_name="s")

    deg2d = pl.kernel(
        _degree_body,
        out_type=jax.ShapeDtypeStruct((NC, NPAD, 16), jnp.float32),
        mesh=mesh,
        compiler_params=pltpu.CompilerParams(use_tc_tiling_on_sc=False),
        scratch_types=[
            pltpu.VMEM_SHARED((NPAD, 16), jnp.float32),
            pltpu.VMEM((CH, 16), jnp.float32),
            pltpu.VMEM((EPW,), jnp.int32),
            pltpu.VMEM((CH,), jnp.int32),
            pltpu.VMEM((CH,), jnp.int32),
            pltpu.SemaphoreType.DMA,
            pltpu.SemaphoreType.DMA,
        ],
    )(edge2, z16)

    nblk = 10
    blk = N_NODES // nblk
    y = pl.pallas_call(
        _prepass_body,
        grid=(nblk,),
        in_specs=[
            pl.BlockSpec((blk, CHANNELS), lambda i: (i, 0)),
            pl.BlockSpec((NC, blk, 16), lambda i: (0, i, 0)),
        ],
        out_specs=pl.BlockSpec((blk, CHANNELS), lambda i: (i, 0)),
        out_shape=jax.ShapeDtypeStruct((N_NODES, CHANNELS), jnp.bfloat16),
    )(x, deg2d)

    props = pl.kernel(
        _aggregate_body,
        out_type=jax.ShapeDtypeStruct((NW, ROWS_PER_SUB, CHANNELS),
                                      jnp.bfloat16),
        mesh=mesh,
        compiler_params=pltpu.CompilerParams(use_tc_tiling_on_sc=False),
        scratch_types=[
            pltpu.VMEM_SHARED((NPAD, CHANNELS), jnp.bfloat16),
            pltpu.VMEM((EPW,), jnp.int32),
            pltpu.VMEM((EPW,), jnp.int32),
            [pltpu.VMEM((CH,), jnp.int32) for _ in range(NSLOT)],
            [pltpu.VMEM((CH, CHANNELS), jnp.bfloat16) for _ in range(NSLOT)],
            [pltpu.SemaphoreType.DMA for _ in range(NSLOT)],
        ],
    )(edge2, y, z128)
    props1d = props.reshape(NC * NPAD * CHANNELS)

    fblk = 1024
    fnblk = NPAD // fblk
    out = pl.pallas_call(
        _final_body,
        grid=(fnblk,),
        in_specs=[
            pl.BlockSpec((fblk, CHANNELS), lambda i: (i, 0)),
            pl.BlockSpec((CHANNELS, CHANNELS), lambda i: (0, 0)),
            pl.BlockSpec((1, CHANNELS), lambda i: (0, 0)),
            pl.BlockSpec((NC, fblk, 16), lambda i: (0, i, 0)),
            pl.BlockSpec((fblk * CHANNELS,), lambda i: (i,)),
            pl.BlockSpec((fblk * CHANNELS,), lambda i: (i + fnblk,)),
        ],
        out_specs=pl.BlockSpec((fblk, CHANNELS), lambda i: (i, 0)),
        out_shape=jax.ShapeDtypeStruct((N_NODES, CHANNELS), jnp.float32),
    )(x, weight, bias2d, deg2d, props1d, props1d)

    return out


# speedup vs baseline: 1.0243x; 1.0243x over previous
import jax
import jax.numpy as jnp
from jax import lax
from jax.experimental import pallas as pl
from jax.experimental.pallas import tpu as pltpu
from jax.experimental.pallas import tpu_sc as plsc

N_NODES = 10000
N_EDGES = 320000
CHANNELS = 128
EPSILON = 0.1
GAMMA = 0.1

NPAD = 10240
NC = 2
NS = 16
NW = NC * NS
CH = 80
EPW = N_EDGES // NW
M = EPW // CH
ROWS_PER_SUB = NPAD // NS


def _copy80(src_ref, dst_ref, base):
    for k in range(CH // 16):
        dst_ref[pl.ds(k * 16, 16)] = src_ref[pl.ds(base + k * 16, 16)]


def _degree_body(edge_hbm, z_hbm, deg_out, deg_sp, ones_v, didx1d,
                 dcur0, dcur1, sem0, sem1):
    c = lax.axis_index("c")
    s = lax.axis_index("s")
    wid = c * NS + s

    @pl.when(s == 0)
    def _():
        pltpu.sync_copy(z_hbm, deg_sp)

    pltpu.sync_copy(edge_hbm.at[NW + wid], didx1d)

    ov = jnp.ones((16,), jnp.float32)

    def fill_ones(i, _):
        ones_v[i, :] = ov
        return 0

    lax.fori_loop(0, CH, fill_ones, 0)
    plsc.subcore_barrier()

    _copy80(didx1d, dcur0, 0)
    pltpu.async_copy(ones_v, deg_sp.at[dcur0], sem0, add=True)
    _copy80(didx1d, dcur1, CH)
    pltpu.async_copy(ones_v, deg_sp.at[dcur1], sem1, add=True)

    def body(jj, _):
        a = 2 * jj
        b = a + 1
        pltpu.make_async_copy(ones_v, deg_sp.at[dcur0], sem0).wait()
        _copy80(didx1d, dcur0, (a + 2) * CH)
        pltpu.async_copy(ones_v, deg_sp.at[dcur0], sem0, add=True)
        pltpu.make_async_copy(ones_v, deg_sp.at[dcur1], sem1).wait()

        @pl.when(b + 2 < M)
        def _():
            _copy80(didx1d, dcur1, (b + 2) * CH)
            pltpu.async_copy(ones_v, deg_sp.at[dcur1], sem1, add=True)

        return 0

    lax.fori_loop(0, (M - 1) // 2, body, 0)
    pltpu.make_async_copy(ones_v, deg_sp.at[dcur0], sem0).wait()
    plsc.subcore_barrier()

    sl = pl.ds(s * ROWS_PER_SUB, ROWS_PER_SUB)
    pltpu.sync_copy(deg_sp.at[sl], deg_out.at[c, sl])


NSLOT = 4


def _aggregate_body(edge_hbm, y_hbm, z_hbm, props_out,
                    acc_sp, sidx1d, didx1d, dcurs, rows, sems):
    c = lax.axis_index("c")
    s = lax.axis_index("s")
    wid = c * NS + s

    @pl.when(s == 0)
    def _():
        pltpu.sync_copy(z_hbm, acc_sp)

    pltpu.sync_copy(edge_hbm.at[wid], sidx1d)
    pltpu.sync_copy(edge_hbm.at[NW + wid], didx1d)
    plsc.subcore_barrier()

    def g_idx(j):
        return sidx1d.at[pl.ds(j * CH, CH)]

    for k in range(NSLOT):
        pltpu.async_copy(y_hbm.at[g_idx(k)], rows[k], sems[k])

    def body(jj, _):
        base4 = NSLOT * jj
        for k in range(NSLOT):
            ch = base4 + k
            _copy80(didx1d, dcurs[k], ch * CH)
            pltpu.make_async_copy(y_hbm.at[g_idx(ch)], rows[k], sems[k]).wait()
            pltpu.sync_copy(rows[k], acc_sp.at[dcurs[k]], add=True)

            @pl.when(ch + NSLOT < M)
            def _():
                pltpu.async_copy(y_hbm.at[g_idx(ch + NSLOT)], rows[k], sems[k])

        return 0

    lax.fori_loop(0, (M - 1) // NSLOT, body, 0)
    _copy80(didx1d, dcurs[0], (M - 1) * CH)
    pltpu.make_async_copy(y_hbm.at[g_idx(M - 1)], rows[0], sems[0]).wait()
    pltpu.sync_copy(rows[0], acc_sp.at[dcurs[0]], add=True)
    plsc.subcore_barrier()

    sl = pl.ds(s * ROWS_PER_SUB, ROWS_PER_SUB)
    pltpu.sync_copy(acc_sp.at[sl], props_out.at[c * NS + s])


def _prepass_body(x_ref, deg_ref, y_ref):
    deg = (deg_ref[0] + deg_ref[1])[:, 0:1]
    dinv = jnp.where(deg > 0.5, lax.rsqrt(jnp.maximum(deg, 1e-12)), 0.0)
    y_ref[...] = (x_ref[...] * dinv).astype(jnp.bfloat16)


def _final_body(x_ref, w_ref, b_ref, deg_ref, p0_ref, p1_ref, out_ref):
    deg = (deg_ref[0] + deg_ref[1])[:, 0:1]
    dinv = jnp.where(deg > 0.5, lax.rsqrt(jnp.maximum(deg, 1e-12)), 0.0)
    blk = out_ref.shape[0]
    prop = (p0_ref[...].astype(jnp.float32).reshape(blk, CHANNELS)
            + p1_ref[...].astype(jnp.float32).reshape(blk, CHANNELS)) * dinv
    w = w_ref[...]
    ii = lax.broadcasted_iota(jnp.int32, (CHANNELS, CHANNELS), 0)
    jj = lax.broadcasted_iota(jnp.int32, (CHANNELS, CHANNELS), 1)
    eye = jnp.where(ii == jj, jnp.float32(GAMMA), jnp.float32(0.0))
    op = w - w.T - eye
    xb = x_ref[...]
    pre = jnp.dot(xb, op, preferred_element_type=jnp.float32) + b_ref[...]
    out_ref[...] = xb + EPSILON * jnp.tanh(pre + prop)


def kernel(x, edge_index, weight, bias):
    edge2 = edge_index.astype(jnp.int32).reshape(2 * NW, EPW)
    bias2d = bias.reshape(1, CHANNELS)
    z16 = jnp.zeros((NPAD, 16), jnp.float32)
    z128 = jnp.zeros((NPAD, CHANNELS), jnp.bfloat16)

    mesh = plsc.VectorSubcoreMesh(core_axis_name="c", subcore_axis_name="s")

    deg2d = pl.kernel(
        _degree_body,
        out_type=jax.ShapeDtypeStruct((NC, NPAD, 16), jnp.float32),
        mesh=mesh,
        compiler_params=pltpu.CompilerParams(use_tc_tiling_on_sc=False),
        scratch_types=[
            pltpu.VMEM_SHARED((NPAD, 16), jnp.float32),
            pltpu.VMEM((CH, 16), jnp.float32),
            pltpu.VMEM((EPW,), jnp.int32),
            pltpu.VMEM((CH,), jnp.int32),
            pltpu.VMEM((CH,), jnp.int32),
            pltpu.SemaphoreType.DMA,
            pltpu.SemaphoreType.DMA,
        ],
    )(edge2, z16)

    nblk = 10
    blk = N_NODES // nblk
    y = pl.pallas_call(
        _prepass_body,
        grid=(nblk,),
        in_specs=[
            pl.BlockSpec((blk, CHANNELS), lambda i: (i, 0)),
            pl.BlockSpec((NC, blk, 16), lambda i: (0, i, 0)),
        ],
        out_specs=pl.BlockSpec((blk, CHANNELS), lambda i: (i, 0)),
        out_shape=jax.ShapeDtypeStruct((N_NODES, CHANNELS), jnp.bfloat16),
    )(x, deg2d)

    props = pl.kernel(
        _aggregate_body,
        out_type=jax.ShapeDtypeStruct((NW, ROWS_PER_SUB, CHANNELS),
                                      jnp.bfloat16),
        mesh=mesh,
        compiler_params=pltpu.CompilerParams(use_tc_tiling_on_sc=False),
        scratch_types=[
            pltpu.VMEM_SHARED((NPAD, CHANNELS), jnp.bfloat16),
            pltpu.VMEM((EPW,), jnp.int32),
            pltpu.VMEM((EPW,), jnp.int32),
            [pltpu.VMEM((CH,), jnp.int32) for _ in range(NSLOT)],
            [pltpu.VMEM((CH, CHANNELS), jnp.bfloat16) for _ in range(NSLOT)],
            [pltpu.SemaphoreType.DMA for _ in range(NSLOT)],
        ],
    )(edge2, y, z128)
    props1d = props.reshape(NC * NPAD * CHANNELS)

    fblk = 1024
    fnblk = NPAD // fblk
    out = pl.pallas_call(
        _final_body,
        grid=(fnblk,),
        in_specs=[
            pl.BlockSpec((fblk, CHANNELS), lambda i: (i, 0)),
            pl.BlockSpec((CHANNELS, CHANNELS), lambda i: (0, 0)),
            pl.BlockSpec((1, CHANNELS), lambda i: (0, 0)),
            pl.BlockSpec((NC, fblk, 16), lambda i: (0, i, 0)),
            pl.BlockSpec((fblk * CHANNELS,), lambda i: (i,)),
            pl.BlockSpec((fblk * CHANNELS,), lambda i: (i + fnblk,)),
        ],
        out_specs=pl.BlockSpec((fblk, CHANNELS), lambda i: (i, 0)),
        out_shape=jax.ShapeDtypeStruct((N_NODES, CHANNELS), jnp.float32),
    )(x, weight, bias2d, deg2d, props1d, props1d)

    return out
